# Initial kernel scaffold; baseline (speedup 1.0000x reference)
import jax
import jax.numpy as jnp
from jax.experimental import pallas as pl


def kernel(x, edge_index, edge_attr, batch, W1, b1, W2, b2, root, bias, Wf1, bf1, Wf2, bf2):
    raise NotImplementedError("write your pallas kernel here")


# speedup vs baseline: 1.6603x; 1.6603x over previous
import functools

import jax
import jax.numpy as jnp
from jax import lax
from jax.experimental import pallas as pl
from jax.experimental.pallas import tpu as pltpu
from jax.experimental.pallas import tpu_sc as plsc

N = 10000
E = 160000
C = 128
ED = 16
H = 8
G = 128
OUTD = 1
HC = H * C

NC = 2
NS = 16
NW = NC * NS
EPAD = 163840
PERW = EPAD // NW
GCHUNK = 128
NGC = PERW // GCHUNK
VCHUNK = 1024
NVC = PERW // VCHUNK
BE = 640
NBLK = EPAD // BE

def _wid():
    return lax.axis_index("s") * NC + lax.axis_index("c")


def _gather_body(x_hbm, src_hbm, out_hbm, idx_v, rows_v, sem):
    base = _wid() * PERW

    def body(i, carry):
        off = base + i * GCHUNK
        pltpu.sync_copy(src_hbm.at[pl.ds(off, GCHUNK)], idx_v)
        pltpu.async_copy(x_hbm.at[idx_v], rows_v, sem).wait()
        pltpu.sync_copy(rows_v, out_hbm.at[pl.ds(off, GCHUNK)])
        return carry

    lax.fori_loop(0, NGC, body, 0)


def _edge_body(ea, xj, w1, b1, w2p, b2p, s2, out):
    h1 = jnp.maximum(
        jnp.dot(ea[...], w1[...], preferred_element_type=jnp.float32) + b1[...],
        0.0)
    th = jnp.dot(h1, w2p[...], preferred_element_type=jnp.float32) + b2p[...]
    xt = jnp.concatenate([xj[...]] * H, axis=1)
    msg = jnp.dot(xt * th, s2[...], preferred_element_type=jnp.float32)
    out[...] = msg.T


def _edge_call(ea_p, xj, w1, b1r, w2p, b2p, s2):
    return pl.pallas_call(
        _edge_body,
        grid=(NBLK,),
        in_specs=[
            pl.BlockSpec((BE, ED), lambda i: (i, 0)),
            pl.BlockSpec((BE, C), lambda i: (i, 0)),
            pl.BlockSpec((ED, HC), lambda i: (0, 0)),
            pl.BlockSpec((1, HC), lambda i: (0, 0)),
            pl.BlockSpec((HC, HC), lambda i: (0, 0)),
            pl.BlockSpec((1, HC), lambda i: (0, 0)),
            pl.BlockSpec((HC, H), lambda i: (0, 0)),
        ],
        out_specs=pl.BlockSpec((H, BE), lambda i: (0, i)),
        out_shape=jax.ShapeDtypeStruct((H, EPAD), jnp.float32),
        compiler_params=pltpu.CompilerParams(
            dimension_semantics=("arbitrary",)),
    )(ea_p, xj, w1, b1r, w2p, b2p, s2)


def _scatter_body(msgt_hbm, dst_hbm, zeros_hbm, out_hbm, acc, mbuf, dbuf):
    w = _wid()
    base = w * PERW
    pltpu.sync_copy(zeros_hbm, acc)

    def chunk_body(ci, carry):
        off = base + ci * VCHUNK
        pltpu.sync_copy(dst_hbm.at[pl.ds(off, VCHUNK)], dbuf)
        for h in range(H):
            pltpu.sync_copy(msgt_hbm.at[h, pl.ds(off, VCHUNK)], mbuf.at[h])

        def vec_body(v, c2):
            dstv = dbuf[pl.ds(v * 16, 16)]
            for h in range(H):
                vals = mbuf[h, pl.ds(v * 16, 16)]
                plsc.addupdate_scatter(acc, (dstv + h * N,), vals)
            return c2

        lax.fori_loop(0, VCHUNK // 16, vec_body, 0)
        return carry

    lax.fori_loop(0, NVC, chunk_body, 0)
    pltpu.sync_copy(acc, out_hbm.at[w])


def _final_body(parts, x, batch, root, bias, wf1t, bf1, wf2t, bf2, out):
    acc_t = parts[0]
    for i in range(1, NW):
        acc_t = acc_t + parts[i]
    xroot_t = lax.dot_general(
        root[...], x[...], (((0,), (1,)), ((), ())),
        preferred_element_type=jnp.float32)
    node_t = jnp.maximum(acc_t + xroot_t + bias[...], 0.0)
    gi = lax.broadcasted_iota(jnp.int32, (N, G), 1)
    oh = jnp.where(gi == batch[...], 1.0, 0.0)
    sums_t = jnp.dot(node_t, oh, preferred_element_type=jnp.float32)
    counts = jnp.sum(oh, axis=0, keepdims=True)
    pooled_t = sums_t / jnp.maximum(counts, 1.0)
    h2_t = jnp.maximum(
        jnp.dot(wf1t[...], pooled_t, preferred_element_type=jnp.float32)
        + bf1[...], 0.0)
    out[...] = (jnp.dot(wf2t[...], h2_t, preferred_element_type=jnp.float32)
                + bf2[...])


def _final_call(parts, x, batch_col, root, bias_col, wf1t, bf1_col, wf2t,
                bf2_col):
    return pl.pallas_call(
        _final_body,
        out_shape=jax.ShapeDtypeStruct((OUTD, G), jnp.float32),
    )(parts, x, batch_col, root, bias_col, wf1t, bf1_col, wf2t, bf2_col)


def kernel(x, edge_index, edge_attr, batch, W1, b1, W2, b2, root, bias,
           Wf1, bf1, Wf2, bf2):
    src = edge_index[0]
    dst = edge_index[1]
    pad = EPAD - E
    x_aug = jnp.concatenate([x, jnp.zeros((8, C), jnp.float32)], axis=0)
    src_p = jnp.concatenate([src, jnp.full((pad,), N, jnp.int32)])
    dst_p = jnp.concatenate([dst, jnp.zeros((pad,), jnp.int32)])
    ea_p = jnp.concatenate(
        [edge_attr, jnp.zeros((pad, ED), jnp.float32)], axis=0)

    w2p = W2.reshape(HC, C, H).transpose(0, 2, 1).reshape(HC, HC)
    b2p = b2.reshape(C, H).T.reshape(1, HC)
    b1r = b1.reshape(1, HC)
    s2 = (jnp.arange(HC, dtype=jnp.int32)[:, None] // C
          == jnp.arange(H, dtype=jnp.int32)[None, :]).astype(jnp.float32)

    mesh = plsc.VectorSubcoreMesh(
        core_axis_name="c", subcore_axis_name="s",
        num_cores=NC, num_subcores=NS)
    gather_k = pl.kernel(
        _gather_body,
        out_type=jax.ShapeDtypeStruct((EPAD, C), jnp.float32),
        mesh=mesh,
        scratch_types=[
            pltpu.VMEM((GCHUNK,), jnp.int32),
            pltpu.VMEM((GCHUNK, C), jnp.float32),
            pltpu.SemaphoreType.DMA,
        ],
    )
    scatter_k = pl.kernel(
        _scatter_body,
        out_type=jax.ShapeDtypeStruct((NW, N * H), jnp.float32),
        mesh=mesh,
        scratch_types=[
            pltpu.VMEM((N * H,), jnp.float32),
            pltpu.VMEM((H, VCHUNK), jnp.float32),
            pltpu.VMEM((VCHUNK,), jnp.int32),
        ],
        compiler_params=pltpu.CompilerParams(needs_layout_passes=False),
    )

    xj = gather_k(x_aug, src_p)
    msgt = _edge_call(ea_p, xj, W1, b1r, w2p, b2p, s2)
    zeros = jnp.zeros((N * H,), jnp.float32)
    parts = scatter_k(msgt, dst_p, zeros).reshape(NW, H, N)
    out_t = _final_call(parts, x, batch.reshape(N, 1), root,
                        bias.reshape(H, 1), Wf1.T, bf1.reshape(H, 1),
                        Wf2.T, bf2.reshape(OUTD, 1))
    return out_t.reshape(G, OUTD)


# speedup vs baseline: 1.7014x; 1.0247x over previous
import functools

import jax
import jax.numpy as jnp
from jax import lax
from jax.experimental import pallas as pl
from jax.experimental.pallas import tpu as pltpu
from jax.experimental.pallas import tpu_sc as plsc

N = 10000
E = 160000
C = 128
ED = 16
H = 8
G = 128
OUTD = 1
HC = H * C

NC = 2
NS = 16
NW = NC * NS
EPAD = 163840
PERW = EPAD // NW
GCHUNK = 128
NGC = PERW // GCHUNK
VCHUNK = 1024
NVC = PERW // VCHUNK
BE = 640
NBLK = EPAD // BE

def _wid():
    return lax.axis_index("s") * NC + lax.axis_index("c")


GBUF = 4
NGG = NGC // GBUF


def _gather_body(x_hbm, src_hbm, out_hbm, idx_v, rows_v, gsem, osem):
    base = _wid() * PERW
    pltpu.sync_copy(src_hbm.at[pl.ds(base, PERW)], idx_v)

    def group(g, carry):
        goff = g * (GBUF * GCHUNK)
        descs = []
        for b in range(GBUF):
            off = goff + b * GCHUNK
            descs.append(pltpu.async_copy(
                x_hbm.at[idx_v.at[pl.ds(off, GCHUNK)]], rows_v.at[b], gsem))
        for d in descs:
            d.wait()
        descs = []
        for b in range(GBUF):
            off = goff + b * GCHUNK
            descs.append(pltpu.async_copy(
                rows_v.at[b], out_hbm.at[pl.ds(base + off, GCHUNK)], osem))
        for d in descs:
            d.wait()
        return carry

    lax.fori_loop(0, NGG, group, 0)


def _edge_body(ea, xj, w1, b1, w2p, b2p, s2, out):
    h1 = jnp.maximum(
        jnp.dot(ea[...], w1[...], preferred_element_type=jnp.float32) + b1[...],
        0.0)
    th = jnp.dot(h1, w2p[...], preferred_element_type=jnp.float32) + b2p[...]
    xt = jnp.concatenate([xj[...]] * H, axis=1)
    msg = jnp.dot(xt * th, s2[...], preferred_element_type=jnp.float32)
    out[...] = msg.T


def _edge_call(ea_p, xj, w1, b1r, w2p, b2p, s2):
    return pl.pallas_call(
        _edge_body,
        grid=(NBLK,),
        in_specs=[
            pl.BlockSpec((BE, ED), lambda i: (i, 0)),
            pl.BlockSpec((BE, C), lambda i: (i, 0)),
            pl.BlockSpec((ED, HC), lambda i: (0, 0)),
            pl.BlockSpec((1, HC), lambda i: (0, 0)),
            pl.BlockSpec((HC, HC), lambda i: (0, 0)),

            pl.BlockSpec((1, HC), lambda i: (0, 0)),
            pl.BlockSpec((HC, H), lambda i: (0, 0)),
        ],
        out_specs=pl.BlockSpec((H, BE), lambda i: (0, i)),
        out_shape=jax.ShapeDtypeStruct((H, EPAD), jnp.float32),
        compiler_params=pltpu.CompilerParams(
            dimension_semantics=("arbitrary",)),
    )(ea_p, xj, w1, b1r, w2p, b2p, s2)


def _scatter_body(msgt_hbm, dst_hbm, zeros_hbm, out_hbm, acc, mbuf, dbuf):
    w = _wid()
    base = w * PERW
    pltpu.sync_copy(zeros_hbm, acc)

    def chunk_body(ci, carry):
        off = base + ci * VCHUNK
        pltpu.sync_copy(dst_hbm.at[pl.ds(off, VCHUNK)], dbuf)
        for h in range(H):
            pltpu.sync_copy(msgt_hbm.at[h, pl.ds(off, VCHUNK)], mbuf.at[h])

        def vec_body(v, c2):
            dstv = dbuf[pl.ds(v * 16, 16)]
            for h in range(H):
                vals = mbuf[h, pl.ds(v * 16, 16)]
                plsc.addupdate_scatter(acc, (dstv + h * N,), vals)
            return c2

        lax.fori_loop(0, VCHUNK // 16, vec_body, 0)
        return carry

    lax.fori_loop(0, NVC, chunk_body, 0)
    pltpu.sync_copy(acc, out_hbm.at[w])


def _final_body(parts, x, batch, root, bias, wf1t, bf1, wf2t, bf2, out):
    acc_t = parts[0]
    for i in range(1, NW):
        acc_t = acc_t + parts[i]
    xroot_t = lax.dot_general(
        root[...], x[...], (((0,), (1,)), ((), ())),
        preferred_element_type=jnp.float32)
    node_t = jnp.maximum(acc_t + xroot_t + bias[...], 0.0)
    gi = lax.broadcasted_iota(jnp.int32, (N, G), 1)
    oh = jnp.where(gi == batch[...], 1.0, 0.0)
    sums_t = jnp.dot(node_t, oh, preferred_element_type=jnp.float32)
    counts = jnp.sum(oh, axis=0, keepdims=True)
    pooled_t = sums_t / jnp.maximum(counts, 1.0)
    h2_t = jnp.maximum(
        jnp.dot(wf1t[...], pooled_t, preferred_element_type=jnp.float32)
        + bf1[...], 0.0)
    out[...] = (jnp.dot(wf2t[...], h2_t, preferred_element_type=jnp.float32)
                + bf2[...])


def _final_call(parts, x, batch_col, root, bias_col, wf1t, bf1_col, wf2t,
                bf2_col):
    return pl.pallas_call(
        _final_body,
        out_shape=jax.ShapeDtypeStruct((OUTD, G), jnp.float32),
    )(parts, x, batch_col, root, bias_col, wf1t, bf1_col, wf2t, bf2_col)


def kernel(x, edge_index, edge_attr, batch, W1, b1, W2, b2, root, bias,
           Wf1, bf1, Wf2, bf2):
    src = edge_index[0]
    dst = edge_index[1]
    pad = EPAD - E
    x_aug = jnp.concatenate([x, jnp.zeros((8, C), jnp.float32)], axis=0)
    src_p = jnp.concatenate([src, jnp.full((pad,), N, jnp.int32)])
    dst_p = jnp.concatenate([dst, jnp.zeros((pad,), jnp.int32)])
    ea_p = jnp.concatenate(
        [edge_attr, jnp.zeros((pad, ED), jnp.float32)], axis=0)

    w2p = W2.reshape(HC, C, H).transpose(0, 2, 1).reshape(HC, HC)
    b2p = b2.reshape(C, H).T.reshape(1, HC)
    b1r = b1.reshape(1, HC)
    s2 = (jnp.arange(HC, dtype=jnp.int32)[:, None] // C
          == jnp.arange(H, dtype=jnp.int32)[None, :]).astype(jnp.float32)

    mesh = plsc.VectorSubcoreMesh(
        core_axis_name="c", subcore_axis_name="s",
        num_cores=NC, num_subcores=NS)
    gather_k = pl.kernel(
        _gather_body,
        out_type=jax.ShapeDtypeStruct((EPAD, C), jnp.float32),
        mesh=mesh,
        scratch_types=[
            pltpu.VMEM((PERW,), jnp.int32),
            pltpu.VMEM((GBUF, GCHUNK, C), jnp.float32),
            pltpu.SemaphoreType.DMA,
            pltpu.SemaphoreType.DMA,
        ],
    )
    scatter_k = pl.kernel(
        _scatter_body,
        out_type=jax.ShapeDtypeStruct((NW, N * H), jnp.float32),
        mesh=mesh,
        scratch_types=[
            pltpu.VMEM((N * H,), jnp.float32),
            pltpu.VMEM((H, VCHUNK), jnp.float32),
            pltpu.VMEM((VCHUNK,), jnp.int32),
        ],
        compiler_params=pltpu.CompilerParams(needs_layout_passes=False),
    )

    xj = gather_k(x_aug, src_p)
    msgt = _edge_call(ea_p, xj, W1, b1r, w2p, b2p, s2)
    zeros = jnp.zeros((N * H,), jnp.float32)
    parts = scatter_k(msgt, dst_p, zeros).reshape(NW, H, N)
    out_t = _final_call(parts, x, batch.reshape(N, 1), root,
                        bias.reshape(H, 1), Wf1.T, bf1.reshape(H, 1),
                        Wf2.T, bf2.reshape(OUTD, 1))
    return out_t.reshape(G, OUTD)


# speedup vs baseline: 1.8170x; 1.0680x over previous
import functools

import jax
import jax.numpy as jnp
from jax import lax
from jax.experimental import pallas as pl
from jax.experimental.pallas import tpu as pltpu
from jax.experimental.pallas import tpu_sc as plsc

N = 10000
E = 160000
C = 128
ED = 16
H = 8
G = 128
OUTD = 1
HC = H * C

NC = 2
NS = 16
NW = NC * NS
EPAD = 163840
NSPLIT = 2
ESP = EPAD // NSPLIT
PERW = ESP // NW
GCHUNK = 128
GBUF = 4
NGG = PERW // (GCHUNK * GBUF)
VCHUNK = 512
NVC = PERW // VCHUNK
BE = 640
NBLK = ESP // BE


def _wid():
    return lax.axis_index("s") * NC + lax.axis_index("c")


def _gather_body(x_hbm, src_hbm, out_hbm, idx_v, rows_v, gsem, osem):
    base = _wid() * PERW
    pltpu.sync_copy(src_hbm.at[pl.ds(base, PERW)], idx_v)

    def group(g, carry):
        goff = g * (GBUF * GCHUNK)
        descs = []
        for b in range(GBUF):
            off = goff + b * GCHUNK
            descs.append(pltpu.async_copy(
                x_hbm.at[idx_v.at[pl.ds(off, GCHUNK)]], rows_v.at[b], gsem))
        for d in descs:
            d.wait()
        descs = []
        for b in range(GBUF):
            off = goff + b * GCHUNK
            descs.append(pltpu.async_copy(
                rows_v.at[b], out_hbm.at[pl.ds(base + off, GCHUNK)], osem))
        for d in descs:
            d.wait()
        return carry

    lax.fori_loop(0, NGG, group, 0)


def _edge_body(ea, xj, w1, b1, w2p, b2p, s2, out):
    h1 = jnp.maximum(
        jnp.dot(ea[...], w1[...], preferred_element_type=jnp.float32) + b1[...],
        0.0)
    th = jnp.dot(h1, w2p[...], preferred_element_type=jnp.float32) + b2p[...]
    xt = jnp.concatenate([xj[...]] * H, axis=1)
    msg = jnp.dot(xt * th, s2[...], preferred_element_type=jnp.float32)
    out[...] = msg.T


def _edge_call(ea_p, xj, w1, b1r, w2p, b2p, s2):
    return pl.pallas_call(
        _edge_body,
        grid=(NBLK,),
        in_specs=[
            pl.BlockSpec((BE, ED), lambda i: (i, 0)),
            pl.BlockSpec((BE, C), lambda i: (i, 0)),
            pl.BlockSpec((ED, HC), lambda i: (0, 0)),
            pl.BlockSpec((1, HC), lambda i: (0, 0)),
            pl.BlockSpec((HC, HC), lambda i: (0, 0)),
            pl.BlockSpec((1, HC), lambda i: (0, 0)),
            pl.BlockSpec((HC, H), lambda i: (0, 0)),
        ],
        out_specs=pl.BlockSpec((H, BE), lambda i: (0, i)),
        out_shape=jax.ShapeDtypeStruct((H, ESP), jnp.float32),
        compiler_params=pltpu.CompilerParams(
            dimension_semantics=("arbitrary",)),
    )(ea_p, xj, w1, b1r, w2p, b2p, s2)


def _scatter_body(msgt_hbm, dst_hbm, zeros_hbm, out_hbm, acc, mbuf, dbuf):
    w = _wid()
    base = w * PERW
    pltpu.sync_copy(zeros_hbm, acc)

    def chunk_body(ci, carry):
        off = base + ci * VCHUNK
        pltpu.sync_copy(dst_hbm.at[pl.ds(off, VCHUNK)], dbuf)
        for h in range(H):
            pltpu.sync_copy(msgt_hbm.at[h, pl.ds(off, VCHUNK)], mbuf.at[h])

        def vec_body(v, c2):
            dstv = dbuf[pl.ds(v * 16, 16)]
            for h in range(H):
                vals = mbuf[h, pl.ds(v * 16, 16)]
                plsc.addupdate_scatter(acc, (dstv + h * N,), vals)
            return c2

        lax.fori_loop(0, VCHUNK // 16, vec_body, 0)
        return carry

    lax.fori_loop(0, NVC, chunk_body, 0)
    pltpu.sync_copy(acc, out_hbm.at[w])


def _final_body(parts, x, batch, root, bias, wf1t, bf1, wf2t, bf2, out):
    acc_t = parts[0]
    for i in range(1, NSPLIT * NW):
        acc_t = acc_t + parts[i]
    xroot_t = lax.dot_general(
        root[...], x[...], (((0,), (1,)), ((), ())),
        preferred_element_type=jnp.float32)
    node_t = jnp.maximum(acc_t + xroot_t + bias[...], 0.0)
    gi = lax.broadcasted_iota(jnp.int32, (N, G), 1)
    oh = jnp.where(gi == batch[...], 1.0, 0.0)
    sums_t = jnp.dot(node_t, oh, preferred_element_type=jnp.float32)
    counts = jnp.sum(oh, axis=0, keepdims=True)
    pooled_t = sums_t / jnp.maximum(counts, 1.0)
    h2_t = jnp.maximum(
        jnp.dot(wf1t[...], pooled_t, preferred_element_type=jnp.float32)
        + bf1[...], 0.0)
    out[...] = (jnp.dot(wf2t[...], h2_t, preferred_element_type=jnp.float32)
                + bf2[...])


def _final_call(parts, x, batch_col, root, bias_col, wf1t, bf1_col, wf2t,
                bf2_col):
    return pl.pallas_call(
        _final_body,
        out_shape=jax.ShapeDtypeStruct((OUTD, G), jnp.float32),
    )(parts, x, batch_col, root, bias_col, wf1t, bf1_col, wf2t, bf2_col)


def kernel(x, edge_index, edge_attr, batch, W1, b1, W2, b2, root, bias,
           Wf1, bf1, Wf2, bf2):
    src = edge_index[0]
    dst = edge_index[1]
    pad = EPAD - E
    x_aug = jnp.concatenate([x, jnp.zeros((8, C), jnp.float32)], axis=0)
    src_p = jnp.concatenate([src, jnp.full((pad,), N, jnp.int32)])
    dst_p = jnp.concatenate([dst, jnp.zeros((pad,), jnp.int32)])
    ea_p = jnp.concatenate(
        [edge_attr, jnp.zeros((pad, ED), jnp.float32)], axis=0)

    w2p = W2.reshape(HC, C, H).transpose(0, 2, 1).reshape(HC, HC)
    b2p = b2.reshape(C, H).T.reshape(1, HC)
    b1r = b1.reshape(1, HC)
    s2 = (jnp.arange(HC, dtype=jnp.int32)[:, None] // C
          == jnp.arange(H, dtype=jnp.int32)[None, :]).astype(jnp.float32)

    mesh = plsc.VectorSubcoreMesh(
        core_axis_name="c", subcore_axis_name="s",
        num_cores=NC, num_subcores=NS)
    gather_k = pl.kernel(
        _gather_body,
        out_type=jax.ShapeDtypeStruct((ESP, C), jnp.float32),
        mesh=mesh,
        scratch_types=[
            pltpu.VMEM((PERW,), jnp.int32),
            pltpu.VMEM((GBUF, GCHUNK, C), jnp.float32),
            pltpu.SemaphoreType.DMA,
            pltpu.SemaphoreType.DMA,
        ],
    )
    scatter_k = pl.kernel(
        _scatter_body,
        out_type=jax.ShapeDtypeStruct((NW, N * H), jnp.float32),
        mesh=mesh,
        scratch_types=[
            pltpu.VMEM((N * H,), jnp.float32),
            pltpu.VMEM((H, VCHUNK), jnp.float32),
            pltpu.VMEM((VCHUNK,), jnp.int32),
        ],
        compiler_params=pltpu.CompilerParams(needs_layout_passes=False),
    )

    zeros = jnp.zeros((N * H,), jnp.float32)
    xjs = []
    for si in range(NSPLIT):
        sl = slice(si * ESP, (si + 1) * ESP)
        xjs.append(gather_k(x_aug, src_p[sl]))
    msgts = []
    for si in range(NSPLIT):
        sl = slice(si * ESP, (si + 1) * ESP)
        msgts.append(_edge_call(ea_p[sl], xjs[si], W1, b1r, w2p, b2p, s2))
    parts = []
    for si in range(NSPLIT):
        sl = slice(si * ESP, (si + 1) * ESP)
        parts.append(scatter_k(msgts[si], dst_p[sl], zeros))
    parts = jnp.concatenate(parts, axis=0).reshape(NSPLIT * NW, H, N)
    out_t = _final_call(parts, x, batch.reshape(N, 1), root,
                        bias.reshape(H, 1), Wf1.T, bf1.reshape(H, 1),
                        Wf2.T, bf2.reshape(OUTD, 1))
    return out_t.reshape(G, OUTD)
